# baseline (speedup 1.0000x reference)
import jax
import jax.numpy as jnp
import numpy as np
from jax import lax
from jax.experimental import pallas as pl
from jax.experimental.pallas import tpu as pltpu
from jax.experimental.pallas import tpu_sc as plsc

N_LEVELS = 16
F = 2
LOG2_T = 19
T = 1 << LOG2_T
MASK = T - 1
BASE_RES = 16
SCALE = 1.5
N = 524288
M_HASH = int(np.uint32(2654435761).view(np.int32))

NC, NS = 2, 16
NW = NC * NS
PT = N // NW
B = 1024
NCH = PT // B
GROUPS = B // 16
NIDX = 4 * B
GCHUNK = 4096
NG = NIDX // GCHUNK

LPG = 2
NGRP = N_LEVELS // LPG
GT = LPG * T
SLICE = GT // NS

RES = [float(np.floor(BASE_RES * (SCALE ** l))) for l in range(N_LEVELS)]
HI16 = int(np.uint32(0xFFFF0000).view(np.int32))


def _hash_grid_body(xy, tbl, out, xyv, wxv, wyv, idxv, fpv, outv, stbl, sem):
    wid = lax.axis_index("s") * NC + lax.axis_index("c")
    sid = lax.axis_index("s")
    iota = lax.iota(jnp.int32, 16)

    def load_xy(base):
        pltpu.sync_copy(xy.at[pl.ds(2 * base, 2 * B)], xyv)

    def idxgen_fire(g, ll, par):
        res = RES[g * LPG + ll]
        kl = ll << 7
        io = par * NIDX
        wo = par * B

        def body(j, c):
            boff = ((j >> 3) << 8) + ((j & 7) << 4)
            posx = xyv[pl.ds(boff, 16)] * res
            posy = xyv[pl.ds(boff + 128, 16)] * res
            ix = posx.astype(jnp.int32)
            iy = posy.astype(jnp.int32)
            wxv[pl.ds(wo + j * 16, 16)] = posx - ix.astype(jnp.float32)
            wyv[pl.ds(wo + j * 16, 16)] = posy - iy.astype(jnp.float32)
            iym = iy * M_HASH
            iym1 = iym + M_HASH
            ix1 = ix + 1
            h00 = (ix ^ iym) & MASK
            h10 = (ix1 ^ iym) & MASK
            h01 = (ix ^ iym1) & MASK
            h11 = (ix1 ^ iym1) & MASK
            idxv[pl.ds(io + j * 16, 16)] = \
                (((h00 >> 7) << 8) + (h00 & 127)) + kl
            idxv[pl.ds(io + B + j * 16, 16)] = \
                (((h10 >> 7) << 8) + (h10 & 127)) + kl
            idxv[pl.ds(io + 2 * B + j * 16, 16)] = \
                (((h01 >> 7) << 8) + (h01 & 127)) + kl
            idxv[pl.ds(io + 3 * B + j * 16, 16)] = \
                (((h11 >> 7) << 8) + (h11 & 127)) + kl
            return c

        lax.fori_loop(0, GROUPS, body, 0)

        def fire(k, c):
            s = pl.ds(io + k * GCHUNK, GCHUNK)
            pltpu.async_copy(stbl.at[idxv.at[s]], fpv.at[s], sem)
            return c

        lax.fori_loop(0, NG, fire, 0)

    def wait_task(par):
        io = par * NIDX
        pltpu.make_async_copy(tbl.at[pl.ds(0, NIDX)],
                              fpv.at[pl.ds(io, NIDX)], sem).wait()

    def combine(ll, par):
        io = par * NIDX
        wo = par * B

        def unpk(v):
            fa = plsc.bitcast(v << 16, jnp.float32)
            fb = plsc.bitcast(v & HI16, jnp.float32)
            return fa, fb

        def body(j, c):
            f00a, f00b = unpk(fpv[pl.ds(io + j * 16, 16)])
            f10a, f10b = unpk(fpv[pl.ds(io + B + j * 16, 16)])
            f01a, f01b = unpk(fpv[pl.ds(io + 2 * B + j * 16, 16)])
            f11a, f11b = unpk(fpv[pl.ds(io + 3 * B + j * 16, 16)])
            wx = wxv[pl.ds(wo + j * 16, 16)]
            wy = wyv[pl.ds(wo + j * 16, 16)]
            u = 1.0 - wx
            v = 1.0 - wy
            w00 = u * v
            w10 = wx * v
            w01 = u * wy
            w11 = wx * wy
            acc_a = w00 * f00a + w10 * f10a + w01 * f01a + w11 * f11a
            acc_b = w00 * f00b + w10 * f10b + w01 * f01b + w11 * f11b
            opos = ((j * 16 + iota) << 2) + 2 * ll
            plsc.store_scatter(outv, [opos], acc_a)
            plsc.store_scatter(outv, [opos + 1], acc_b)
            return c

        lax.fori_loop(0, GROUPS, body, 0)

    for g in range(NGRP):
        plsc.subcore_barrier()
        pltpu.sync_copy(tbl.at[pl.ds(g * GT + sid * SLICE, SLICE)],
                        stbl.at[pl.ds(sid * SLICE, SLICE)])
        plsc.subcore_barrier()

        load_xy(wid * PT)
        idxgen_fire(g, 0, 0)

        def chunk_body(ci, carry, g=g):
            base = wid * PT + ci * B
            for ll in range(LPG - 1):
                idxgen_fire(g, ll + 1, (ll + 1) % 2)
                wait_task(ll % 2)
                combine(ll, ll % 2)

            @pl.when(ci < NCH - 1)
            def _():
                load_xy(base + B)
                idxgen_fire(g, 0, 0)

            wait_task((LPG - 1) % 2)
            combine(LPG - 1, (LPG - 1) % 2)
            pltpu.sync_copy(
                outv, out.at[pl.ds(g * (N * 2 * LPG) + base * 2 * LPG,
                                   B * 2 * LPG)])
            return carry

        lax.fori_loop(0, NCH, chunk_body, 0)


@jax.jit
def _hash_grid_sc(xy, tbl):
    mesh = plsc.VectorSubcoreMesh(core_axis_name="c", subcore_axis_name="s",
                                  num_cores=NC, num_subcores=NS)
    return pl.kernel(
        _hash_grid_body,
        out_type=jax.ShapeDtypeStruct((N * 2 * N_LEVELS,), jnp.float32),
        mesh=mesh,
        compiler_params=pltpu.CompilerParams(needs_layout_passes=False),
        scratch_types=[
            pltpu.VMEM((2 * B,), jnp.float32),
            pltpu.VMEM((2 * B,), jnp.float32),
            pltpu.VMEM((2 * B,), jnp.float32),
            pltpu.VMEM((2 * NIDX,), jnp.int32),
            pltpu.VMEM((2 * NIDX,), jnp.int32),
            pltpu.VMEM((B * 2 * LPG,), jnp.float32),
            pltpu.VMEM_SHARED((GT,), jnp.int32),
            pltpu.SemaphoreType.DMA,
        ],
    )(xy, tbl)


def kernel(x, table):
    xy = x.reshape(N // 128, 128, 2).transpose(0, 2, 1).reshape(2 * N)
    xi = lax.bitcast_convert_type(table, jnp.int32)
    packed = ((xi[:, :, 0] >> 16) & 0xFFFF) | (xi[:, :, 1] & HI16)
    tbl = (packed.reshape(NGRP, LPG, T // 128, 128)
           .transpose(0, 2, 1, 3)
           .reshape(N_LEVELS * T))
    out = _hash_grid_sc(xy, tbl)
    return (out.reshape(NGRP, N, 2 * LPG)
            .transpose(1, 0, 2)
            .reshape(N, 2 * N_LEVELS))


# speedup vs baseline: 1.2038x; 1.2038x over previous
import jax
import jax.numpy as jnp
import numpy as np
from jax import lax
from jax.experimental import pallas as pl
from jax.experimental.pallas import tpu as pltpu
from jax.experimental.pallas import tpu_sc as plsc

N_LEVELS = 16
F = 2
LOG2_T = 19
T = 1 << LOG2_T
MASK = T - 1
BASE_RES = 16
SCALE = 1.5
N = 524288
M_HASH = int(np.uint32(2654435761).view(np.int32))

NC, NS = 2, 16
NW = NC * NS
PT = N // NW
B = 1024
NCH = PT // B
GROUPS = B // 16
NIDX = 4 * B
GCHUNK = 4096
NG = NIDX // GCHUNK

RES = [float(np.floor(BASE_RES * (SCALE ** l))) for l in range(N_LEVELS)]
LBASE = [((l >> 3) << 22) + ((l & 7) << 7) for l in range(N_LEVELS)]
HI16 = int(np.uint32(0xFFFF0000).view(np.int32))


def _hash_grid_body(xy, tbl, out, xyv, wxv, wyv, idxv, fpv, outv, sem):
    wid = lax.axis_index("s") * NC + lax.axis_index("c")
    iota = lax.iota(jnp.int32, 16)

    def load_xy(base):
        pltpu.sync_copy(xy.at[pl.ds(2 * base, 2 * B)], xyv)

    def idxgen_fire(l, par):
        res = RES[l]
        lb = LBASE[l]
        io = par * NIDX
        wo = par * B

        def body(j, c):
            boff = ((j >> 3) << 8) + ((j & 7) << 4)
            posx = xyv[pl.ds(boff, 16)] * res
            posy = xyv[pl.ds(boff + 128, 16)] * res
            ix = posx.astype(jnp.int32)
            iy = posy.astype(jnp.int32)
            wxv[pl.ds(wo + j * 16, 16)] = posx - ix.astype(jnp.float32)
            wyv[pl.ds(wo + j * 16, 16)] = posy - iy.astype(jnp.float32)
            iym = iy * M_HASH
            iym1 = iym + M_HASH
            ix1 = ix + 1
            h00 = (ix ^ iym) & MASK
            h10 = (ix1 ^ iym) & MASK
            h01 = (ix ^ iym1) & MASK
            h11 = (ix1 ^ iym1) & MASK
            idxv[pl.ds(io + j * 16, 16)] = \
                (((h00 >> 7) << 10) + (h00 & 127)) + lb
            idxv[pl.ds(io + B + j * 16, 16)] = \
                (((h10 >> 7) << 10) + (h10 & 127)) + lb
            idxv[pl.ds(io + 2 * B + j * 16, 16)] = \
                (((h01 >> 7) << 10) + (h01 & 127)) + lb
            idxv[pl.ds(io + 3 * B + j * 16, 16)] = \
                (((h11 >> 7) << 10) + (h11 & 127)) + lb
            return c

        lax.fori_loop(0, GROUPS, body, 0)

        def fire(g, c):
            s = pl.ds(io + g * GCHUNK, GCHUNK)
            pltpu.async_copy(tbl.at[idxv.at[s]], fpv.at[s], sem)
            return c

        lax.fori_loop(0, NG, fire, 0)

    def wait_task(par):
        io = par * NIDX
        pltpu.make_async_copy(tbl.at[pl.ds(0, NIDX)],
                              fpv.at[pl.ds(io, NIDX)], sem).wait()

    def combine(l, par):
        io = par * NIDX
        wo = par * B

        def unpk(v):
            fa = plsc.bitcast(v << 16, jnp.float32)
            fb = plsc.bitcast(v & HI16, jnp.float32)
            return fa, fb

        def body(j, c):
            f00a, f00b = unpk(fpv[pl.ds(io + j * 16, 16)])
            f10a, f10b = unpk(fpv[pl.ds(io + B + j * 16, 16)])
            f01a, f01b = unpk(fpv[pl.ds(io + 2 * B + j * 16, 16)])
            f11a, f11b = unpk(fpv[pl.ds(io + 3 * B + j * 16, 16)])
            wx = wxv[pl.ds(wo + j * 16, 16)]
            wy = wyv[pl.ds(wo + j * 16, 16)]
            u = 1.0 - wx
            v = 1.0 - wy
            w00 = u * v
            w10 = wx * v
            w01 = u * wy
            w11 = wx * wy
            acc_a = w00 * f00a + w10 * f10a + w01 * f01a + w11 * f11a
            acc_b = w00 * f00b + w10 * f10b + w01 * f01b + w11 * f11b
            opos = ((j * 16 + iota) << 5) + 2 * l
            plsc.store_scatter(outv, [opos], acc_a)
            plsc.store_scatter(outv, [opos + 1], acc_b)
            return c

        lax.fori_loop(0, GROUPS, body, 0)

    load_xy(wid * PT)
    idxgen_fire(0, 0)

    def chunk_body(ci, carry):
        base = wid * PT + ci * B
        for l in range(N_LEVELS - 1):
            idxgen_fire(l + 1, (l + 1) % 2)
            wait_task(l % 2)
            combine(l, l % 2)

        @pl.when(ci < NCH - 1)
        def _():
            load_xy(base + B)
            idxgen_fire(0, 0)

        wait_task((N_LEVELS - 1) % 2)
        combine(N_LEVELS - 1, (N_LEVELS - 1) % 2)
        pltpu.sync_copy(outv, out.at[pl.ds(base * 2 * N_LEVELS,
                                           B * 2 * N_LEVELS)])
        return carry

    lax.fori_loop(0, NCH, chunk_body, 0)


@jax.jit
def _hash_grid_sc(xy, tbl):
    mesh = plsc.VectorSubcoreMesh(core_axis_name="c", subcore_axis_name="s",
                                  num_cores=NC, num_subcores=NS)
    return pl.kernel(
        _hash_grid_body,
        out_type=jax.ShapeDtypeStruct((N * 2 * N_LEVELS,), jnp.float32),
        mesh=mesh,
        compiler_params=pltpu.CompilerParams(needs_layout_passes=False),
        scratch_types=[
            pltpu.VMEM((2 * B,), jnp.float32),
            pltpu.VMEM((2 * B,), jnp.float32),
            pltpu.VMEM((2 * B,), jnp.float32),
            pltpu.VMEM((2 * NIDX,), jnp.int32),
            pltpu.VMEM((2 * NIDX,), jnp.int32),
            pltpu.VMEM((B * 2 * N_LEVELS,), jnp.float32),
            pltpu.SemaphoreType.DMA,
        ],
    )(xy, tbl)


def kernel(x, table):
    xy = x.reshape(N // 128, 128, 2).transpose(0, 2, 1).reshape(2 * N)
    xi = lax.bitcast_convert_type(table, jnp.int32)
    packed = ((xi[:, :, 0] >> 16) & 0xFFFF) | (xi[:, :, 1] & HI16)
    tbl = (packed.reshape(2, 8, T // 128, 128)
           .transpose(0, 2, 1, 3)
           .reshape(N_LEVELS * T))
    return _hash_grid_sc(xy, tbl).reshape(N, 2 * N_LEVELS)
